# baseline (speedup 1.0000x reference)
import functools

import jax
import jax.numpy as jnp
import numpy as np
from jax import lax
from jax.experimental import pallas as pl
from jax.experimental.pallas import tpu as pltpu
from jax.experimental.pallas import tpu_sc as plsc

_F = 26
_D = 16
_BATCH = 4096
_FIELD = 100000

_NC = 2
_NS = 16
_NW = _NC * _NS
_BPW = _BATCH // _NW

_OFFSETS = np.arange(_F, dtype=np.int32) * _FIELD


def _gather_body(tab_hbm, idx_hbm, s_hbm, q_hbm, idx_v, rows_v, s_v, q_v, sem):
    wid = lax.axis_index("s") * _NC + lax.axis_index("c")

    pltpu.sync_copy(idx_hbm.at[wid], idx_v)

    copies = []
    for j in range(_F):
        copies.append(
            pltpu.async_copy(tab_hbm.at[idx_v.at[j]], rows_v.at[j], sem))
    for c in copies:
        c.wait()

    def b_body(b, carry):
        v = rows_v[0, b]
        s = v
        q = v * v
        for j in range(1, _F):
            v = rows_v[j, b]
            s = s + v
            q = q + v * v
        s_v[b] = s
        q_v[b] = q
        return carry

    lax.fori_loop(0, _BPW, b_body, 0)

    pltpu.sync_copy(s_v, s_hbm.at[pl.ds(wid * _BPW, _BPW)])
    pltpu.sync_copy(q_v, q_hbm.at[pl.ds(wid * _BPW, _BPW)])


_gather_kernel = functools.partial(
    pl.kernel,
    out_type=(
        jax.ShapeDtypeStruct((_BATCH, _D), jnp.float32),
        jax.ShapeDtypeStruct((_BATCH, _D), jnp.float32),
    ),
    mesh=plsc.VectorSubcoreMesh(core_axis_name="c", subcore_axis_name="s"),
    compiler_params=pltpu.CompilerParams(use_tc_tiling_on_sc=False),
    scratch_types=[
        pltpu.VMEM((_F, _BPW), jnp.int32),
        pltpu.VMEM((_F, _BPW, _D), jnp.float32),
        pltpu.VMEM((_BPW, _D), jnp.float32),
        pltpu.VMEM((_BPW, _D), jnp.float32),
        pltpu.SemaphoreType.DMA,
    ],
)(_gather_body)


def _finalize_body(s_ref, q_ref, o_ref):
    s = s_ref[...]
    q = q_ref[...]
    ix = 0.5 * jnp.sum(s * s - q, axis=1)
    o_ref[...] = jax.nn.sigmoid(ix)


_finalize_kernel = pl.pallas_call(
    _finalize_body,
    out_shape=jax.ShapeDtypeStruct((_BATCH,), jnp.float32),
)


def kernel(x, table):
    idx = x.astype(jnp.int32) + jnp.asarray(_OFFSETS)[None, :]
    idx3 = idx.reshape(_NW, _BPW, _F).transpose(0, 2, 1)
    sp, qp = _gather_kernel(table, idx3)
    return _finalize_kernel(sp, qp)


# speedup vs baseline: 2.6004x; 2.6004x over previous
import functools

import jax
import jax.numpy as jnp
import numpy as np
from jax import lax
from jax.experimental import pallas as pl
from jax.experimental.pallas import tpu as pltpu
from jax.experimental.pallas import tpu_sc as plsc

_F = 26
_D = 16
_BATCH = 4096
_FIELD = 100000
_COLS = _F * _FIELD

_NC = 2
_NS = 16
_NW = _NC * _NS

_C = 8192
_NFULL = _COLS // _C
_SCAN_END = (_COLS // 128) * 128
_TAIL = _SCAN_END - _NFULL * _C
_TAILW = _NFULL % _NS
_NLAST = _COLS - _SCAN_END

_G = _BATCH // 16

_OFFSETS = np.arange(_F, dtype=np.int32) * _FIELD


def _scan_body(tab_hbm, idx_hbm, sp_hbm, qp_hbm, chunk_v, s_v, q_v, xg_v):
    c = lax.axis_index("c")
    t = lax.axis_index("s")
    wid = c * _NS + t

    zero = jnp.zeros((16,), jnp.float32)

    def init_body(g, carry):
        sl = pl.ds(g * 16, 16)
        for d in range(8):
            s_v[d, sl] = zero
        q_v[sl] = zero
        return carry

    lax.fori_loop(0, _G, init_body, 0)

    def serve(base, size):
        def serve_field(f):
            pltpu.sync_copy(
                idx_hbm.at[pl.ds(pl.multiple_of(f * _BATCH, 8), _BATCH)],
                xg_v)

            def group_body(g, carry):
                sl = pl.ds(g * 16, 16)
                g16 = xg_v[sl]
                col = g16 - base
                m = (col >= 0) & (col < size)
                colc = jnp.minimum(jnp.maximum(col, 0), size - 1)
                q16 = q_v[sl]
                for d in range(8):
                    v = plsc.load_gather(
                        chunk_v, [jnp.full((16,), d, jnp.int32), colc])
                    v = jnp.where(m, v, 0.0)
                    s_v[d, sl] = s_v[d, sl] + v
                    q16 = q16 + v * v
                q_v[sl] = q16
                return carry

            lax.fori_loop(0, _G, group_body, 0)

        f0 = base // _FIELD
        f1 = (base + size - 1) // _FIELD
        serve_field(f0)

        @pl.when(f1 != f0)
        def _():
            serve_field(f1)

    n_chunks = (_NFULL - t + _NS - 1) // _NS

    def chunk_body(i, carry):
        k = t + i * _NS
        base = pl.multiple_of(k * _C, 128)
        pltpu.sync_copy(
            tab_hbm.at[pl.ds(pl.multiple_of(c * 8, 8), 8), pl.ds(base, _C)],
            chunk_v)
        serve(base, _C)
        return carry

    lax.fori_loop(0, n_chunks, chunk_body, 0)

    @pl.when(t == _TAILW)
    def _():
        base = _NFULL * _C
        pltpu.sync_copy(
            tab_hbm.at[pl.ds(pl.multiple_of(c * 8, 8), 8),
                       pl.ds(base, _TAIL)],
            chunk_v.at[:, pl.ds(0, _TAIL)])
        serve(base, _TAIL)

    pltpu.sync_copy(s_v, sp_hbm.at[wid])
    pltpu.sync_copy(q_v, qp_hbm.at[wid])


_scan_kernel = functools.partial(
    pl.kernel,
    out_type=(
        jax.ShapeDtypeStruct((_NW, 8, _BATCH), jnp.float32),
        jax.ShapeDtypeStruct((_NW, _BATCH), jnp.float32),
    ),
    mesh=plsc.VectorSubcoreMesh(core_axis_name="c", subcore_axis_name="s"),
    compiler_params=pltpu.CompilerParams(needs_layout_passes=False),
    scratch_types=[
        pltpu.VMEM((8, _C), jnp.float32),
        pltpu.VMEM((8, _BATCH), jnp.float32),
        pltpu.VMEM((_BATCH,), jnp.float32),
        pltpu.VMEM((_BATCH,), jnp.int32),
    ],
)(_scan_body)


def _finalize_body(sp_ref, qp_ref, tail_ref, idxt_ref, o_ref):
    s0 = jnp.sum(sp_ref[0:_NS], axis=0)
    s1 = jnp.sum(sp_ref[_NS:_NW], axis=0)
    s = jnp.concatenate([s0, s1], axis=0)
    idxt = idxt_ref[...]
    onehot = jnp.where(
        lax.broadcasted_iota(jnp.int32, (_NLAST, _BATCH), 0) == idxt[None, :],
        1.0, 0.0)
    tail = tail_ref[...]
    s = s + jnp.dot(tail, onehot, preferred_element_type=jnp.float32)
    rs = jnp.sum(tail * tail, axis=0)
    q_tail = jnp.sum(onehot * rs[:, None], axis=0)
    ssq = jnp.sum(s * s, axis=0)
    q = jnp.sum(qp_ref[...], axis=0) + q_tail
    o_ref[...] = jax.nn.sigmoid(0.5 * (ssq - q))


_finalize_kernel = pl.pallas_call(
    _finalize_body,
    out_shape=jax.ShapeDtypeStruct((_BATCH,), jnp.float32),
)


def kernel(x, table):
    idx = x.astype(jnp.int32) + jnp.asarray(_OFFSETS)[None, :]
    idx_flat = idx.T.reshape(-1)
    sp, qp = _scan_kernel(table.T, idx_flat)
    tail = table.T[:, _SCAN_END:]
    idxt = idx[:, _F - 1] - _SCAN_END
    return _finalize_kernel(sp, qp, tail, idxt)


# speedup vs baseline: 2.9306x; 1.1270x over previous
import functools

import jax
import jax.numpy as jnp
import numpy as np
from jax import lax
from jax.experimental import pallas as pl
from jax.experimental.pallas import tpu as pltpu
from jax.experimental.pallas import tpu_sc as plsc

_F = 26
_D = 16
_BATCH = 4096
_FIELD = 100000
_COLS = _F * _FIELD

_NC = 2
_NS = 16
_NW = _NC * _NS

_C = 10240
_NFULL = _COLS // _C
_SCAN_END = (_COLS // 128) * 128
_TAIL = _SCAN_END - _NFULL * _C
_TAILW = _NFULL % _NS
_NLAST = _COLS - _SCAN_END

_G = _BATCH // 16

_OFFSETS = np.arange(_F, dtype=np.int32) * _FIELD


def _scan_body(tab_hbm, idx_hbm, sp_hbm, qp_hbm, chunk_v, s_v, q_v, xg_v):
    c = lax.axis_index("c")
    t = lax.axis_index("s")
    wid = c * _NS + t

    zero = jnp.zeros((16,), jnp.float32)

    def init_body(g, carry):
        sl = pl.ds(g * 16, 16)
        for d in range(8):
            s_v[d, sl] = zero
        q_v[sl] = zero
        return carry

    lax.fori_loop(0, _G, init_body, 0)

    def serve(base, size):
        def serve_field(f):
            pltpu.sync_copy(
                idx_hbm.at[pl.ds(pl.multiple_of(f * _BATCH, 8), _BATCH)],
                xg_v)

            def group_body(g, carry):
                sl = pl.ds(g * 16, 16)
                g16 = xg_v[sl]
                col = g16 - base
                m = (col >= 0) & (col < size)

                @pl.when(jnp.any(m))
                def _():
                    colc = jnp.minimum(jnp.maximum(col, 0), size - 1)
                    q16 = q_v[sl]
                    for d in range(8):
                        v = plsc.load_gather(
                            chunk_v, [jnp.full((16,), d, jnp.int32), colc])
                        v = jnp.where(m, v, 0.0)
                        s_v[d, sl] = s_v[d, sl] + v
                        q16 = q16 + v * v
                    q_v[sl] = q16

                return carry

            lax.fori_loop(0, _G, group_body, 0)

        f0 = base // _FIELD
        f1 = (base + size - 1) // _FIELD
        serve_field(f0)

        @pl.when(f1 != f0)
        def _():
            serve_field(f1)

    n_chunks = (_NFULL - t + _NS - 1) // _NS

    def chunk_body(i, carry):
        k = t + i * _NS
        base = pl.multiple_of(k * _C, 128)
        pltpu.sync_copy(
            tab_hbm.at[pl.ds(pl.multiple_of(c * 8, 8), 8), pl.ds(base, _C)],
            chunk_v)
        serve(base, _C)
        return carry

    lax.fori_loop(0, n_chunks, chunk_body, 0)

    @pl.when(t == _TAILW)
    def _():
        base = _NFULL * _C
        pltpu.sync_copy(
            tab_hbm.at[pl.ds(pl.multiple_of(c * 8, 8), 8),
                       pl.ds(base, _TAIL)],
            chunk_v.at[:, pl.ds(0, _TAIL)])
        serve(base, _TAIL)

    pltpu.sync_copy(s_v, sp_hbm.at[wid])
    pltpu.sync_copy(q_v, qp_hbm.at[wid])


_scan_kernel = functools.partial(
    pl.kernel,
    out_type=(
        jax.ShapeDtypeStruct((_NW, 8, _BATCH), jnp.float32),
        jax.ShapeDtypeStruct((_NW, _BATCH), jnp.float32),
    ),
    mesh=plsc.VectorSubcoreMesh(core_axis_name="c", subcore_axis_name="s"),
    compiler_params=pltpu.CompilerParams(needs_layout_passes=False),
    scratch_types=[
        pltpu.VMEM((8, _C), jnp.float32),
        pltpu.VMEM((8, _BATCH), jnp.float32),
        pltpu.VMEM((_BATCH,), jnp.float32),
        pltpu.VMEM((_BATCH,), jnp.int32),
    ],
)(_scan_body)


def _finalize_body(sp_ref, qp_ref, tail_ref, idxt_ref, o_ref):
    s0 = jnp.sum(sp_ref[0:_NS], axis=0)
    s1 = jnp.sum(sp_ref[_NS:_NW], axis=0)
    s = jnp.concatenate([s0, s1], axis=0)
    idxt = idxt_ref[...]
    onehot = jnp.where(
        lax.broadcasted_iota(jnp.int32, (_NLAST, _BATCH), 0) == idxt[None, :],
        1.0, 0.0)
    tail = tail_ref[...]
    s = s + jnp.dot(tail, onehot, preferred_element_type=jnp.float32)
    rs = jnp.sum(tail * tail, axis=0)
    q_tail = jnp.sum(onehot * rs[:, None], axis=0)
    ssq = jnp.sum(s * s, axis=0)
    q = jnp.sum(qp_ref[...], axis=0) + q_tail
    o_ref[...] = jax.nn.sigmoid(0.5 * (ssq - q))


_finalize_kernel = pl.pallas_call(
    _finalize_body,
    out_shape=jax.ShapeDtypeStruct((_BATCH,), jnp.float32),
)


def kernel(x, table):
    idx = x.astype(jnp.int32) + jnp.asarray(_OFFSETS)[None, :]
    idx_flat = idx.T.reshape(-1)
    sp, qp = _scan_kernel(table.T, idx_flat)
    tail = table.T[:, _SCAN_END:]
    idxt = idx[:, _F - 1] - _SCAN_END
    return _finalize_kernel(sp, qp, tail, idxt)
